# baseline (speedup 1.0000x reference)
import functools

import jax
import jax.numpy as jnp
from jax import lax
from jax.experimental import pallas as pl
from jax.experimental.pallas import tpu as pltpu
from jax.experimental.pallas import tpu_sc as plsc

NUM_CORES = 2
NUM_SUBCORES = 16
NUM_WORKERS = NUM_CORES * NUM_SUBCORES

VOCAB = 1000000
BATCH = 16384
EMBED_DIM = 64
PAIR_DIM = 2 * EMBED_DIM
ROWS_PER_WORKER = BATCH // NUM_WORKERS
CHUNK = 128
NUM_CHUNKS = ROWS_PER_WORKER // CHUNK
L = 16
GROUPS_PER_CHUNK = CHUNK // L

TBLK = 16384
K_SC = 25 * TBLK
H2 = 19 * TBLK

_mesh = plsc.VectorSubcoreMesh(core_axis_name="c", subcore_axis_name="s")


def _pair_transpose_body(x1_ref, x2_ref, o_ref):
    o_ref[...] = jnp.concatenate([x1_ref[...], x2_ref[...]], axis=0).T


def _pair_transpose(table_t):
    kb = K_SC // TBLK
    last = VOCAB // TBLK
    return pl.pallas_call(
        _pair_transpose_body,
        out_shape=jax.ShapeDtypeStruct((H2, PAIR_DIM), jnp.float32),
        grid=(H2 // TBLK,),
        in_specs=[
            pl.BlockSpec((EMBED_DIM, TBLK), lambda i: (0, i + kb)),
            pl.BlockSpec(
                (EMBED_DIM, TBLK),
                lambda i: (0, jnp.minimum(i + kb + H2 // TBLK, last)),
            ),
        ],
        out_specs=pl.BlockSpec((TBLK, PAIR_DIM), lambda i: (i, 0)),
    )(table_t, table_t)


@functools.partial(
    pl.kernel,
    mesh=_mesh,
    out_type=jax.ShapeDtypeStruct((BATCH, EMBED_DIM), jnp.float32),
    scratch_types=[
        pltpu.VMEM((ROWS_PER_WORKER,), jnp.int32),
        pltpu.VMEM((NUM_CHUNKS, CHUNK), jnp.int32),
        pltpu.VMEM((NUM_CHUNKS, CHUNK), jnp.int32),
        pltpu.VMEM((CHUNK, PAIR_DIM), jnp.float32),
        pltpu.VMEM((CHUNK, PAIR_DIM), jnp.float32),
        pltpu.VMEM((ROWS_PER_WORKER, EMBED_DIM), jnp.float32),
        pltpu.SemaphoreType.DMA,
    ],
)
def _gather_kernel(
    idx_hbm, sc_hbm, tc_hbm, out_hbm,
    idx_v, pid_sc_v, pid_tc_v, buf_sc, buf_tc, out_v, sem,
):
    wid = lax.axis_index("s") * NUM_CORES + lax.axis_index("c")
    base = wid * ROWS_PER_WORKER
    pltpu.sync_copy(idx_hbm.at[pl.ds(base, ROWS_PER_WORKER)], idx_v)

    n_sc = K_SC // 2

    def pid_body(g, carry):
        vec = idx_v[pl.ds(g * L, L)]
        c = g // GROUPS_PER_CHUNK
        o = (g % GROUPS_PER_CHUNK) * L
        p_sc = jnp.where(vec < K_SC, vec >> 1, 0)
        q = vec - K_SC
        p_tc = jnp.where(vec >= K_SC, jnp.where(q >= H2, q - H2, q), 0)
        pid_sc_v[c, pl.ds(o, L)] = p_sc
        pid_tc_v[c, pl.ds(o, L)] = p_tc
        return carry

    lax.fori_loop(0, ROWS_PER_WORKER // L, pid_body, 0, unroll=True)

    for c in range(NUM_CHUNKS):
        cp1 = pltpu.async_copy(sc_hbm.at[pid_sc_v.at[c]], buf_sc, sem)
        cp2 = pltpu.async_copy(tc_hbm.at[pid_tc_v.at[c]], buf_tc, sem)
        cp1.wait()
        cp2.wait()

        def sel_body(g, carry):
            vec = idx_v[pl.ds(c * CHUNK + g * L, L)]
            q = vec - K_SC
            off_tc = jnp.where(q >= H2, EMBED_DIM, 0)
            off_sc = (vec & 1) * EMBED_DIM
            off_vec = jnp.where(vec >= K_SC, off_tc, off_sc)
            tc_vec = jnp.where(vec >= K_SC, 1, 0)
            for k in range(L):
                j = g * L + k
                off = off_vec[k]
                use_tc = tc_vec[k]

                @pl.when(use_tc == 1)
                def _():
                    for q_ in range(EMBED_DIM // L):
                        out_v[c * CHUNK + j, pl.ds(q_ * L, L)] = buf_tc[
                            j, pl.ds(off + q_ * L, L)
                        ]

                @pl.when(use_tc == 0)
                def _():
                    for q_ in range(EMBED_DIM // L):
                        out_v[c * CHUNK + j, pl.ds(q_ * L, L)] = buf_sc[
                            j, pl.ds(off + q_ * L, L)
                        ]

            return carry

        lax.fori_loop(0, GROUPS_PER_CHUNK, sel_body, 0)

    pltpu.sync_copy(out_v, out_hbm.at[pl.ds(base, ROWS_PER_WORKER)])


def kernel(prompt_id, table):
    idx = prompt_id.astype(jnp.int32)
    pairs_sc = table[:K_SC].reshape(K_SC // 2, PAIR_DIM)
    pairs_tc = _pair_transpose(table.T)
    return _gather_kernel(idx, pairs_sc, pairs_tc)


# speedup vs baseline: 4.0852x; 4.0852x over previous
import functools

import jax
import jax.numpy as jnp
from jax import lax
from jax.experimental import pallas as pl
from jax.experimental.pallas import tpu as pltpu
from jax.experimental.pallas import tpu_sc as plsc

NUM_CORES = 2
NUM_SUBCORES = 16
NUM_WORKERS = NUM_CORES * NUM_SUBCORES

VOCAB = 1000000
BATCH = 16384
EMBED_DIM = 64
PAIR_DIM = 2 * EMBED_DIM
ROWS_PER_WORKER = BATCH // NUM_WORKERS
CHUNK = 128
NUM_CHUNKS = ROWS_PER_WORKER // CHUNK
L = 16
GROUPS_PER_CHUNK = CHUNK // L

TBLK = 16384
HALF = 31 * TBLK

_mesh = plsc.VectorSubcoreMesh(core_axis_name="c", subcore_axis_name="s")


def _pair_transpose_body(x1_ref, x2_ref, o_ref):
    o_ref[...] = jnp.concatenate([x1_ref[...], x2_ref[...]], axis=0).T


def _pair_transpose(table_t):
    return pl.pallas_call(
        _pair_transpose_body,
        out_shape=jax.ShapeDtypeStruct((HALF, PAIR_DIM), jnp.float32),
        grid=(HALF // TBLK,),
        in_specs=[
            pl.BlockSpec((EMBED_DIM, TBLK), lambda i: (0, i)),
            pl.BlockSpec(
                (EMBED_DIM, TBLK),
                lambda i: (0, jnp.minimum(i + HALF // TBLK, VOCAB // TBLK)),
            ),
        ],
        out_specs=pl.BlockSpec((TBLK, PAIR_DIM), lambda i: (i, 0)),
    )(table_t, table_t)


@functools.partial(
    pl.kernel,
    mesh=_mesh,
    out_type=jax.ShapeDtypeStruct((BATCH, EMBED_DIM), jnp.float32),
    scratch_types=[
        pltpu.VMEM((ROWS_PER_WORKER,), jnp.int32),
        pltpu.VMEM((NUM_CHUNKS, CHUNK), jnp.int32),
        pltpu.VMEM((2, CHUNK, PAIR_DIM), jnp.float32),
        pltpu.VMEM((ROWS_PER_WORKER, EMBED_DIM), jnp.float32),
        pltpu.SemaphoreType.DMA,
    ],
)
def _gather_kernel(idx_hbm, table_hbm, out_hbm, idx_v, pid_v, pairs_v, out_v, sem):
    wid = lax.axis_index("s") * NUM_CORES + lax.axis_index("c")
    base = wid * ROWS_PER_WORKER
    pltpu.sync_copy(idx_hbm.at[pl.ds(base, ROWS_PER_WORKER)], idx_v)

    def pid_body(g, carry):
        vec = idx_v[pl.ds(g * L, L)]
        pid_v[g // GROUPS_PER_CHUNK, pl.ds((g % GROUPS_PER_CHUNK) * L, L)] = (
            jnp.where(vec >= HALF, vec - HALF, vec)
        )
        return carry

    lax.fori_loop(0, ROWS_PER_WORKER // L, pid_body, 0, unroll=True)

    def fire(c):
        return pltpu.async_copy(
            table_hbm.at[pid_v.at[c]], pairs_v.at[c % 2], sem
        )

    def select(c):
        def sel_body(g, carry):
            vec = idx_v[pl.ds(c * CHUNK + g * L, L)]
            off_vec = jnp.where(vec >= HALF, EMBED_DIM, 0)
            for k in range(L):
                j = g * L + k
                off = off_vec[k]
                for q in range(EMBED_DIM // L):
                    out_v[c * CHUNK + j, pl.ds(q * L, L)] = pairs_v[
                        c % 2, j, pl.ds(off + q * L, L)
                    ]
            return carry

        lax.fori_loop(0, GROUPS_PER_CHUNK, sel_body, 0)

    copies = [fire(0)]
    for c in range(NUM_CHUNKS):
        if c + 1 < NUM_CHUNKS:
            copies.append(fire(c + 1))
        copies[c].wait()
        select(c)

    pltpu.sync_copy(out_v, out_hbm.at[pl.ds(base, ROWS_PER_WORKER)])


def kernel(prompt_id, table):
    idx = prompt_id.astype(jnp.int32)
    table_pairs = _pair_transpose(table.T)
    return _gather_kernel(idx, table_pairs)
